# baseline (speedup 1.0000x reference)
import functools
import jax
import jax.numpy as jnp
from jax import lax
from jax.experimental import pallas as pl
from jax.experimental.pallas import tpu as pltpu
from jax.experimental.pallas import tpu_sc as plsc

_SEQ, _BATCH, _D = 200, 4096, 32
_TOTAL = _SEQ * _BATCH
_V = 1_000_000
_NC, _NS, _L = 2, 16, 16
_NW = _NC * _NS
_SCALE = 8.0
_DP = _D + 1

_mesh = plsc.VectorSubcoreMesh(core_axis_name="c", subcore_axis_name="s")

_WCOL = 512
_NWIN = 1954
_VPAD = _NWIN * _WCOL
_W_BASE = _NWIN // _NW
_W_EXTRA = _NWIN % _NW
_W_MAIN = _W_BASE - 1


@functools.partial(
    pl.kernel,
    out_type=jax.ShapeDtypeStruct((_VPAD * _DP,), jnp.float32),
    mesh=_mesh,
    scratch_types=[
        [pltpu.VMEM((_D, _WCOL + 1), jnp.float32) for _ in range(2)],
        [pltpu.VMEM((_WCOL * _DP,), jnp.float32) for _ in range(2)],
        [pltpu.SemaphoreType.DMA for _ in range(2)],
        [pltpu.SemaphoreType.DMA for _ in range(2)],
    ],
    compiler_params=pltpu.CompilerParams(
        use_tc_tiling_on_sc=True, needs_layout_passes=False
    ),
)
def _format_table(wt_hbm, tab_hbm, tbufs, stages, isems, osems):
    wid = lax.axis_index("s") * _NC + lax.axis_index("c")
    nwin = _W_BASE + jnp.where(wid < _W_EXTRA, 1, 0)
    win0 = wid * _W_BASE + jnp.minimum(wid, _W_EXTRA)

    def col0(i):
        return pl.multiple_of((win0 + i) * _WCOL, 128)

    def tile_in(i, b):
        return pltpu.make_async_copy(
            wt_hbm.at[:, pl.ds(col0(i), _WCOL)],
            tbufs[b].at[:, pl.ds(0, _WCOL)], isems[b]
        )

    def row_out(i, b):
        return pltpu.make_async_copy(
            stages[b], tab_hbm.at[pl.ds(col0(i) * _DP, _WCOL * _DP)], osems[b]
        )

    iota = lax.iota(jnp.int32, _L)
    rowsel = [iota + h * _L for h in range(2)]

    def transpose_scale(b):
        @plsc.parallel_loop(0, _WCOL, unroll=4)
        def _tr(c):
            csplat = jnp.broadcast_to(c, (_L,)).astype(jnp.int32)
            for h in range(2):
                v = plsc.load_gather(tbufs[b], [rowsel[h], csplat])
                stages[b][pl.ds(c * _DP + h * _L, _L)] = v * _SCALE

    for b in range(2):
        tile_in(b, b).start()

    @pl.loop(0, _W_MAIN, step=2)
    def _win(g0):
        for b in range(2):
            g = g0 + b
            tile_in(g, b).wait()

            @pl.when(g0 >= 2)
            def _():
                row_out(g - 2, b).wait()

            transpose_scale(b)

            @pl.when(g + 2 < nwin)
            def _():
                tile_in(g + 2, b).start()

            row_out(g, b).start()

    tile_in(_W_MAIN, 0).wait()
    row_out(_W_MAIN - 2, 0).wait()
    transpose_scale(0)
    row_out(_W_MAIN, 0).start()

    @pl.when(nwin > _W_BASE)
    def _tail():
        tile_in(_W_BASE, 1).wait()
        row_out(_W_MAIN - 1, 1).wait()
        transpose_scale(1)
        row_out(_W_BASE, 1).start()
        row_out(_W_BASE, 1).wait()

    @pl.when(nwin <= _W_BASE)
    def _notail():
        row_out(_W_MAIN - 1, 1).wait()

    row_out(_W_MAIN, 0).wait()


_CHUNK = 512
_CPW = _TOTAL // _CHUNK // _NW
_PER_W = _TOTAL // _NW
_SLABS = _BATCH // 128


@functools.partial(
    pl.kernel,
    out_type=jax.ShapeDtypeStruct((_SEQ, 4, _SLABS * 1024), jnp.float32),
    mesh=_mesh,
    scratch_types=[
        pltpu.VMEM((_PER_W,), jnp.int32),
        [pltpu.VMEM((_CHUNK, _DP), jnp.float32) for _ in range(2)],
        [pltpu.VMEM((4, 4 * 1024), jnp.float32) for _ in range(2)],
        [pltpu.SemaphoreType.DMA for _ in range(2)],
        [pltpu.SemaphoreType.DMA for _ in range(2)],
    ],
    compiler_params=pltpu.CompilerParams(
        use_tc_tiling_on_sc=False, needs_layout_passes=False
    ),
)
def _lookup(idx_hbm, tab_hbm, out_hbm, idx_v, rows, obufs, gsems, osems):
    wid = lax.axis_index("s") * _NC + lax.axis_index("c")
    base = wid * _PER_W
    c0 = wid * _CPW

    pltpu.sync_copy(idx_hbm.at[pl.ds(base, _PER_W)], idx_v)

    def gather(g, b):
        src = tab_hbm.at[idx_v.at[pl.ds(g * _CHUNK, _CHUNK)]]
        return pltpu.make_async_copy(src, rows[b], gsems[b])

    def writeback(g, b):
        cc = c0 + g
        s = cc // 8
        boff = (cc % 8) * (4 * 1024)
        return pltpu.make_async_copy(
            obufs[b], out_hbm.at[s, :, pl.ds(boff, 4 * 1024)], osems[b]
        )

    iota = lax.iota(jnp.int32, _L)
    bsel = [[iota + sl * 128 + q * _L for q in range(8)] for sl in range(4)]

    def transpose(b):
        @plsc.parallel_loop(0, _D, unroll=2)
        def _tr(d):
            dsplat = jnp.broadcast_to(d, (_L,)).astype(jnp.int32)
            dt = d // 8
            roff = (d % 8) * 128
            for sl in range(4):
                for q in range(8):
                    v = plsc.load_gather(rows[b], [bsel[sl][q], dsplat])
                    obufs[b][dt, pl.ds(roff + sl * 1024 + q * _L, _L)] = v

    for b in range(2):
        gather(b, b).start()

    @pl.loop(0, _CPW, step=2)
    def _grp(gg0):
        for b in range(2):
            g = gg0 + b
            gather(g, b).wait()

            @pl.when(gg0 >= 2)
            def _():
                writeback(g - 2, b).wait()

            transpose(b)

            @pl.when(gg0 < _CPW - 2)
            def _():
                gather(g + 2, b).start()

            writeback(g, b).start()

    for b in range(2):
        writeback(_CPW - 2 + b, b).wait()


def kernel(src, weight):
    tab = _format_table(weight.T)
    flat = src.reshape(_TOTAL)
    out3 = _lookup(flat, tab.reshape(_VPAD, _DP))
    out5 = out3.reshape(_SEQ, 4, _SLABS, 8, 128)
    return out5.transpose(0, 2, 4, 1, 3).reshape(_SEQ, _BATCH, _D)


# speedup vs baseline: 9.2967x; 9.2967x over previous
import functools
import jax
import jax.numpy as jnp
from jax import lax
from jax.experimental import pallas as pl
from jax.experimental.pallas import tpu as pltpu
from jax.experimental.pallas import tpu_sc as plsc

_SEQ, _BATCH, _D = 200, 4096, 32
_TOTAL = _SEQ * _BATCH
_V = 1_000_000
_NC, _NS, _L = 2, 16, 16
_NW = _NC * _NS
_SCALE = 8.0
_TP = 513
_RP = 33

_mesh = plsc.VectorSubcoreMesh(core_axis_name="c", subcore_axis_name="s")

_WCOL = 512
_NWIN = 1954
_VPAD = _NWIN * _WCOL
_W_BASE = _NWIN // _NW
_W_EXTRA = _NWIN % _NW
_W_MAIN = _W_BASE - 1


@functools.partial(
    pl.kernel,
    out_type=jax.ShapeDtypeStruct((_VPAD * _D,), jnp.float32),
    mesh=_mesh,
    scratch_types=[
        [pltpu.VMEM((_D, _WCOL), jnp.float32) for _ in range(2)],
        pltpu.VMEM((_D * _TP,), jnp.float32),
        [pltpu.VMEM((_WCOL * _D,), jnp.float32) for _ in range(2)],
        [pltpu.SemaphoreType.DMA for _ in range(2)],
        [pltpu.SemaphoreType.DMA for _ in range(2)],
    ],
    compiler_params=pltpu.CompilerParams(
        use_tc_tiling_on_sc=True, needs_layout_passes=False
    ),
)
def _format_table(wt_hbm, tab_hbm, tbufs, tpad, stages, isems, osems):
    wid = lax.axis_index("s") * _NC + lax.axis_index("c")
    nwin = _W_BASE + jnp.where(wid < _W_EXTRA, 1, 0)
    win0 = wid * _W_BASE + jnp.minimum(wid, _W_EXTRA)

    def col0(i):
        return pl.multiple_of((win0 + i) * _WCOL, 128)

    def tile_in(i, b):
        return pltpu.make_async_copy(
            wt_hbm.at[:, pl.ds(col0(i), _WCOL)], tbufs[b], isems[b]
        )

    def row_out(i, b):
        return pltpu.make_async_copy(
            stages[b], tab_hbm.at[pl.ds(col0(i) * _D, _WCOL * _D)], osems[b]
        )

    iota = lax.iota(jnp.int32, _L)
    gbase = [(iota + h * _L) * _TP for h in range(2)]

    def transpose_scale(b):
        @plsc.parallel_loop(0, _D, unroll=2)
        def _cp(d):
            doff = d * _TP
            for q in range(_WCOL // _L):
                tpad[pl.ds(doff + q * _L, _L)] = tbufs[b][d, pl.ds(q * _L, _L)]

        @plsc.parallel_loop(0, _WCOL, unroll=4)
        def _tr(c):
            csplat = jnp.broadcast_to(c, (_L,)).astype(jnp.int32)
            for h in range(2):
                v = plsc.load_gather(tpad, [gbase[h] + csplat])
                stages[b][pl.ds(c * _D + h * _L, _L)] = v * _SCALE

    for b in range(2):
        tile_in(b, b).start()

    @pl.loop(0, _W_MAIN, step=2)
    def _win(g0):
        for b in range(2):
            g = g0 + b
            tile_in(g, b).wait()

            @pl.when(g0 >= 2)
            def _():
                row_out(g - 2, b).wait()

            transpose_scale(b)

            @pl.when(g + 2 < nwin)
            def _():
                tile_in(g + 2, b).start()

            row_out(g, b).start()

    tile_in(_W_MAIN, 0).wait()
    row_out(_W_MAIN - 2, 0).wait()
    transpose_scale(0)
    row_out(_W_MAIN, 0).start()

    @pl.when(nwin > _W_BASE)
    def _tail():
        tile_in(_W_BASE, 1).wait()
        row_out(_W_MAIN - 1, 1).wait()
        transpose_scale(1)
        row_out(_W_BASE, 1).start()
        row_out(_W_BASE, 1).wait()

    @pl.when(nwin <= _W_BASE)
    def _notail():
        row_out(_W_MAIN - 1, 1).wait()

    row_out(_W_MAIN, 0).wait()


_CHUNK = 512
_CPW = _TOTAL // _CHUNK // _NW
_PER_W = _TOTAL // _NW
_SLABS = _BATCH // 128


@functools.partial(
    pl.kernel,
    out_type=jax.ShapeDtypeStruct((_SEQ, 4, _SLABS * 1024), jnp.float32),
    mesh=_mesh,
    scratch_types=[
        pltpu.VMEM((_PER_W,), jnp.int32),
        [pltpu.VMEM((_CHUNK, _D), jnp.float32) for _ in range(2)],
        pltpu.VMEM((_CHUNK * _RP,), jnp.float32),
        [pltpu.VMEM((4, 4 * 1024), jnp.float32) for _ in range(2)],
        [pltpu.SemaphoreType.DMA for _ in range(2)],
        [pltpu.SemaphoreType.DMA for _ in range(2)],
    ],
    compiler_params=pltpu.CompilerParams(
        use_tc_tiling_on_sc=False, needs_layout_passes=False
    ),
)
def _lookup(idx_hbm, tab_hbm, out_hbm, idx_v, rows, rpad, obufs, gsems, osems):
    wid = lax.axis_index("s") * _NC + lax.axis_index("c")
    base = wid * _PER_W
    c0 = wid * _CPW

    pltpu.sync_copy(idx_hbm.at[pl.ds(base, _PER_W)], idx_v)

    def gather(g, b):
        src = tab_hbm.at[idx_v.at[pl.ds(g * _CHUNK, _CHUNK)]]
        return pltpu.make_async_copy(src, rows[b], gsems[b])

    def writeback(g, b):
        cc = c0 + g
        s = cc // 8
        boff = (cc % 8) * (4 * 1024)
        return pltpu.make_async_copy(
            obufs[b], out_hbm.at[s, :, pl.ds(boff, 4 * 1024)], osems[b]
        )

    iota = lax.iota(jnp.int32, _L)
    bsel = [
        [(iota + sl * 128 + q * _L) * _RP for q in range(8)] for sl in range(4)
    ]

    def transpose(b):
        @plsc.parallel_loop(0, _CHUNK, unroll=4)
        def _cp(r):
            roff = r * _RP
            for h in range(2):
                rpad[pl.ds(roff + h * _L, _L)] = rows[b][r, pl.ds(h * _L, _L)]

        @plsc.parallel_loop(0, _D, unroll=2)
        def _tr(d):
            dsplat = jnp.broadcast_to(d, (_L,)).astype(jnp.int32)
            dt = d // 8
            roff = (d % 8) * 128
            for sl in range(4):
                for q in range(8):
                    v = plsc.load_gather(rpad, [bsel[sl][q] + dsplat])
                    obufs[b][dt, pl.ds(roff + sl * 1024 + q * _L, _L)] = v

    for b in range(2):
        gather(b, b).start()

    @pl.loop(0, _CPW, step=2)
    def _grp(gg0):
        for b in range(2):
            g = gg0 + b
            gather(g, b).wait()

            @pl.when(gg0 >= 2)
            def _():
                writeback(g - 2, b).wait()

            transpose(b)

            @pl.when(gg0 < _CPW - 2)
            def _():
                gather(g + 2, b).start()

            writeback(g, b).start()

    for b in range(2):
        writeback(_CPW - 2 + b, b).wait()


def kernel(src, weight):
    tab = _format_table(weight.T)
    flat = src.reshape(_TOTAL)
    out3 = _lookup(flat, tab.reshape(_VPAD, _D))
    out5 = out3.reshape(_SEQ, 4, _SLABS, 8, 128)
    return out5.transpose(0, 2, 4, 1, 3).reshape(_SEQ, _BATCH, _D)
